# baseline (speedup 1.0000x reference)
import jax
import jax.numpy as jnp
from jax import lax
from jax.experimental import pallas as pl
from jax.experimental.pallas import tpu as pltpu
from jax.experimental.pallas import tpu_sc as plsc

NC = 2
NS = 16
NW = NC * NS
EB = 128


def _ru(x, m):
    return (x + m - 1) // m * m


def _sc_segment_sums(tables, srcs, dsts, n_dst_pads, splits, zeros_f,
                     nbuf=2):
    nspec = len(tables)
    k0s = [s[0] for s in splits]
    k1s = [s[1] for s in splits]
    kmax = max(s.shape[1] for s in srcs)
    W = srcs[0].shape[2]

    out_types = [jax.ShapeDtypeStruct((NC, p, 128), jnp.float32)
                 for p in n_dst_pads]
    scratch = [pltpu.VMEM_SHARED((p, 128), jnp.float32) for p in n_dst_pads]
    scratch += [
        pltpu.VMEM((kmax, W), jnp.int32),
        pltpu.VMEM((kmax, W), jnp.int32),
        pltpu.VMEM((nbuf, W, 128), jnp.float32),
    ]
    scratch += [pltpu.SemaphoreType.DMA] * (2 * nbuf)
    mesh = plsc.VectorSubcoreMesh(core_axis_name="c", subcore_axis_name="s")

    def body(*refs):
        it = iter(refs)
        t_refs = [next(it) for _ in range(nspec)]
        s_refs = [next(it) for _ in range(nspec)]
        d_refs = [next(it) for _ in range(nspec)]
        zf_ref = next(it)
        o_refs = [next(it) for _ in range(nspec)]
        a_refs = [next(it) for _ in range(nspec)]
        sidx = next(it)
        didx = next(it)
        rows = next(it)
        gsem = [next(it) for _ in range(nbuf)]
        ssem = [next(it) for _ in range(nbuf)]

        cid = lax.axis_index("c")
        sid = lax.axis_index("s")
        wid = sid * NC + cid

        for a, p in zip(a_refs, n_dst_pads):
            st = p // NS
            pltpu.sync_copy(zf_ref.at[pl.ds(0, st), :],
                            a.at[pl.ds(sid * st, st), :])
        plsc.subcore_barrier()

        for si in range(nspec):
            kmx = max(k0s[si], k1s[si])
            pltpu.sync_copy(s_refs[si].at[wid], sidx.at[pl.ds(0, kmx), :])
            pltpu.sync_copy(d_refs[si].at[wid], didx.at[pl.ds(0, kmx), :])

            def run(k, si=si):
                gd = [None] * nbuf
                sd = [None] * nbuf
                for j in range(min(nbuf, k)):
                    gd[j] = pltpu.async_copy(t_refs[si].at[sidx.at[j]],
                                             rows.at[j], gsem[j])
                for j in range(k):
                    b = j % nbuf
                    gd[b].wait()
                    sd[b] = pltpu.async_copy(rows.at[b],
                                             a_refs[si].at[didx.at[j]],
                                             ssem[b], add=True)
                    jn = j + nbuf
                    if jn < k:
                        sd[b].wait()
                        gd[b] = pltpu.async_copy(
                            t_refs[si].at[sidx.at[jn]], rows.at[b], gsem[b])
                for j in range(max(0, k - nbuf), k):
                    sd[j % nbuf].wait()

            if k0s[si] == k1s[si]:
                run(k0s[si])
            else:
                @pl.when(cid == 0)
                def _(si=si):
                    run(k0s[si])

                @pl.when(cid != 0)
                def _(si=si):
                    run(k1s[si])
        plsc.subcore_barrier()

        for a, o, p in zip(a_refs, o_refs, n_dst_pads):
            st = p // NS
            pltpu.sync_copy(a.at[pl.ds(sid * st, st), :],
                            o.at[cid, pl.ds(sid * st, st), :])

    fn = pl.kernel(body, out_type=tuple(out_types), mesh=mesh,
                   scratch_types=scratch)
    return fn(*tables, *srcs, *dsts, zeros_f)


def _sc_segment_counts(dsts, n_dst_pads, zeros_f, ones_b):
    nspec = len(dsts)
    ks = [d.shape[1] for d in dsts]
    kmax = max(ks)
    W = dsts[0].shape[2]

    out_types = [jax.ShapeDtypeStruct((NC, p, 128), jnp.float32)
                 for p in n_dst_pads]
    scratch = [pltpu.VMEM_SHARED((p, 128), jnp.float32) for p in n_dst_pads]
    scratch += [
        pltpu.VMEM((nspec, kmax, W), jnp.int32),
        pltpu.VMEM((W, 128), jnp.float32),
        pltpu.SemaphoreType.DMA,
    ]
    mesh = plsc.VectorSubcoreMesh(core_axis_name="c", subcore_axis_name="s")

    def body(*refs):
        it = iter(refs)
        d_refs = [next(it) for _ in range(nspec)]
        zf_ref = next(it)
        on_ref = next(it)
        c_refs = [next(it) for _ in range(nspec)]
        ca_refs = [next(it) for _ in range(nspec)]
        didx = next(it)
        ones = next(it)
        sem = next(it)

        cid = lax.axis_index("c")
        sid = lax.axis_index("s")
        wid = sid * NC + cid

        for ca, p in zip(ca_refs, n_dst_pads):
            st = p // NS
            pltpu.sync_copy(zf_ref.at[pl.ds(0, st), :],
                            ca.at[pl.ds(sid * st, st), :])
        pltpu.sync_copy(on_ref, ones)
        plsc.subcore_barrier()

        for si in range(nspec):
            k = ks[si]
            pltpu.sync_copy(d_refs[si].at[wid],
                            didx.at[si, pl.ds(0, k), :])
        descs = []
        for si in range(nspec):
            for j in range(ks[si]):
                descs.append(pltpu.async_copy(
                    ones, ca_refs[si].at[didx.at[si, j]], sem, add=True))
        for d in descs:
            d.wait()
        plsc.subcore_barrier()

        for ca, co, p in zip(ca_refs, c_refs, n_dst_pads):
            st = p // NS
            pltpu.sync_copy(ca.at[pl.ds(sid * st, st), :],
                            co.at[cid, pl.ds(sid * st, st), :])

    fn = pl.kernel(body, out_type=tuple(out_types), mesh=mesh,
                   scratch_types=scratch)
    return fn(*dsts, zeros_f, ones_b)


def _pad_edges(ei, n_dst, f0=0.5):
    ne = ei.shape[1]
    ne_pad = _ru(ne, NW * EB)
    rtot = ne_pad // EB
    per16 = rtot // NS
    k0 = min(per16 - 1, max(1, int(round(per16 * f0))))
    k1 = per16 - k0
    kmax = max(k0, k1)

    def shape(x, fill):
        xp = jnp.concatenate(
            [x, jnp.full((ne_pad - ne,), fill, jnp.int32)]).reshape(rtot, EB)
        a0 = xp[:NS * k0].reshape(NS, k0, EB)
        a1 = xp[NS * k0:].reshape(NS, k1, EB)
        pad0 = jnp.full((NS, kmax - k0, EB), fill, jnp.int32)
        pad1 = jnp.full((NS, kmax - k1, EB), fill, jnp.int32)
        a0 = jnp.concatenate([a0, pad0], axis=1)
        a1 = jnp.concatenate([a1, pad1], axis=1)
        return jnp.stack([a0, a1], axis=1).reshape(NW, kmax, EB)

    return shape(ei[0], 0), shape(ei[1], n_dst), (k0, k1)


def _tc_matmul(A, W, b, bo=512):
    M, K = A.shape
    O = W.shape[1]
    bo = min(bo, O)

    def mm(a_ref, w_ref, b_ref, o_ref):
        o_ref[...] = jnp.dot(a_ref[...], w_ref[...],
                             preferred_element_type=jnp.float32) + b_ref[...]

    return pl.pallas_call(
        mm,
        grid=(pl.cdiv(O, bo),),
        in_specs=[pl.BlockSpec((M, K), lambda j: (0, 0)),
                  pl.BlockSpec((K, bo), lambda j: (0, j)),
                  pl.BlockSpec((1, bo), lambda j: (0, j))],
        out_specs=pl.BlockSpec((M, bo), lambda j: (0, j)),
        out_shape=jax.ShapeDtypeStruct((M, O), jnp.float32),
    )(A, W, b)


def _tc_combine(S_list, C_list, extra, n_out, pack):
    nsp = len(S_list)

    def body(*refs):
        it = iter(refs)
        Ss = [next(it) for _ in range(nsp)]
        Cs = [next(it) for _ in range(nsp)]
        e_ref = next(it)
        o_ref = next(it)
        acc = None
        for i, (S, C) in enumerate(zip(Ss, Cs)):
            sv = S[0] + S[1]
            cv = C[0, :, 0:1] + C[1, :, 0:1]
            mean = (sv / jnp.maximum(cv, 1.0))[:n_out, :]
            if pack:
                o_ref[:, 128 * i:128 * (i + 1)] = mean
            else:
                acc = mean if acc is None else acc + mean
        if pack:
            o_ref[:, 128 * nsp:] = e_ref[...]
        else:
            o_ref[...] = acc + e_ref[...]

    ow = 128 * (nsp + 1) if pack else 128
    return pl.pallas_call(
        body,
        out_shape=jax.ShapeDtypeStruct((n_out, ow), jnp.float32),
    )(*S_list, *C_list, extra)


def kernel(x_ticker, x_institution, x_mutual_fund, x_news,
           ei_hi, ei_hm, ei_ant, ei_rhm, ei_rhi, ei_rant,
           e1_hi_Wl, e1_hi_bl, e1_hi_Wr, d_hi_Wl, d_hi_bl, d_hi_Wr,
           e1_hm_Wl, e1_hm_bl, e1_hm_Wr, d_hm_Wl, d_hm_bl, d_hm_Wr,
           e1_ant_Wl, e1_ant_bl, e1_ant_Wr, d_ant_Wl, d_ant_bl, d_ant_Wr,
           e1_rhm_Wl, e1_rhm_bl, e1_rhm_Wr, d_rhm_Wl, d_rhm_bl, d_rhm_Wr,
           e1_rhi_Wl, e1_rhi_bl, e1_rhi_Wr, d_rhi_Wl, d_rhi_bl, d_rhi_Wr,
           e1_rant_Wl, e1_rant_bl, e1_rant_Wr,
           d_rant_Wl, d_rant_bl, d_rant_Wr):
    n_tic, n_inst, n_mf, n_news = (x_ticker.shape[0], x_institution.shape[0],
                                   x_mutual_fund.shape[0], x_news.shape[0])
    p_tic = _ru(n_tic + 1, NS * 8)
    p_inst = _ru(n_inst + 1, NS * 8)
    p_mf = _ru(n_mf + 1, NS * 8)
    p_news = _ru(n_news + 1, NS * 8)
    max_st = max(p_tic, p_inst, p_mf, p_news) // NS

    zeros_f = jnp.zeros((_ru(max_st, 8), 128), jnp.float32)
    ones_b = jnp.ones((EB, 128), jnp.float32)

    z384 = jnp.zeros((384,), jnp.float32)
    z128 = jnp.zeros((128,), jnp.float32)
    Pt = _tc_matmul(
        x_ticker,
        jnp.concatenate([e1_hi_Wl.T, e1_hm_Wl.T, e1_rant_Wl.T,
                         (e1_ant_Wr + e1_rhm_Wr + e1_rhi_Wr).T], axis=1),
        jnp.concatenate([z384, e1_ant_bl + e1_rhm_bl + e1_rhi_bl])[None])
    P_hi, P_hm, P_rant, base_tic = (Pt[:, :128], Pt[:, 128:256],
                                    Pt[:, 256:384], Pt[:, 384:])
    Pi = _tc_matmul(x_institution,
                    jnp.concatenate([e1_rhi_Wl.T, e1_hi_Wr.T], axis=1),
                    jnp.concatenate([z128, e1_hi_bl])[None])
    P_rhi, base_inst = Pi[:, :128], Pi[:, 128:]
    Pm = _tc_matmul(x_mutual_fund,
                    jnp.concatenate([e1_rhm_Wl.T, e1_hm_Wr.T], axis=1),
                    jnp.concatenate([z128, e1_hm_bl])[None])
    P_rhm, base_mf = Pm[:, :128], Pm[:, 128:]
    Pn = _tc_matmul(x_news,
                    jnp.concatenate([e1_ant_Wl.T, e1_rant_Wr.T], axis=1),
                    jnp.concatenate([z128, e1_rant_bl])[None])
    P_ant, base_news = Pn[:, :128], Pn[:, 128:]

    F0 = 0.38
    s_hi, d_hi, sp_hi = _pad_edges(ei_hi, n_inst, F0)
    s_hm, d_hm, sp_hm = _pad_edges(ei_hm, n_mf, F0)
    s_ant, d_ant, sp_ant = _pad_edges(ei_ant, n_tic, F0)
    s_rhm, d_rhm, sp_rhm = _pad_edges(ei_rhm, n_tic, F0)
    s_rhi, d_rhi, sp_rhi = _pad_edges(ei_rhi, n_tic, F0)
    s_rant, d_rant, sp_rant = _pad_edges(ei_rant, n_news, F0)

    (C_hi, C_hm, C_ant) = _sc_segment_counts(
        [d_hi, d_hm, d_ant], [p_inst, p_mf, p_tic], zeros_f, ones_b)
    (C_rhm, C_rhi, C_rant) = _sc_segment_counts(
        [d_rhm, d_rhi, d_rant], [p_tic, p_tic, p_news], zeros_f, ones_b)
    (S_hi, S_hm, S_ant) = _sc_segment_sums(
        [P_hi, P_hm, P_ant], [s_hi, s_hm, s_ant], [d_hi, d_hm, d_ant],
        [p_inst, p_mf, p_tic], [sp_hi, sp_hm, sp_ant], zeros_f, nbuf=2)
    (S_rhm, S_rhi, S_rant) = _sc_segment_sums(
        [P_rhm, P_rhi, P_rant], [s_rhm, s_rhi, s_rant],
        [d_rhm, d_rhi, d_rant], [p_tic, p_tic, p_news],
        [sp_rhm, sp_rhi, sp_rant], zeros_f, nbuf=2)

    z_inst = _tc_combine([S_hi], [C_hi], base_inst, n_inst, False)
    z_mf = _tc_combine([S_hm], [C_hm], base_mf, n_mf, False)
    z_tic = _tc_combine([S_ant, S_rhm, S_rhi], [C_ant, C_rhm, C_rhi],
                        base_tic, n_tic, False)
    z_news = _tc_combine([S_rant], [C_rant], base_news, n_news, False)

    (Sd_hi, Sd_hm, Sd_ant) = _sc_segment_sums(
        [z_tic, z_tic, z_news], [s_hi, s_hm, s_ant], [d_hi, d_hm, d_ant],
        [p_inst, p_mf, p_tic], [sp_hi, sp_hm, sp_ant], zeros_f, nbuf=2)
    (Sd_rhm, Sd_rhi, Sd_rant) = _sc_segment_sums(
        [z_mf, z_inst, z_tic], [s_rhm, s_rhi, s_rant],
        [d_rhm, d_rhi, d_rant], [p_tic, p_tic, p_news],
        [sp_rhm, sp_rhi, sp_rant], zeros_f, nbuf=2)

    A_inst = _tc_combine([Sd_hi], [C_hi], z_inst, n_inst, True)
    A_mf = _tc_combine([Sd_hm], [C_hm], z_mf, n_mf, True)
    A_tic = _tc_combine([Sd_ant, Sd_rhm, Sd_rhi], [C_ant, C_rhm, C_rhi],
                        z_tic, n_tic, True)
    A_news = _tc_combine([Sd_rant], [C_rant], z_news, n_news, True)

    out_inst = _tc_matmul(
        A_inst, jnp.concatenate([d_hi_Wl, d_hi_Wr], axis=1).T,
        d_hi_bl[None])
    out_mf = _tc_matmul(
        A_mf, jnp.concatenate([d_hm_Wl, d_hm_Wr], axis=1).T,
        d_hm_bl[None])
    out_tic = _tc_matmul(
        A_tic,
        jnp.concatenate([d_ant_Wl, d_rhm_Wl, d_rhi_Wl,
                         d_ant_Wr + d_rhm_Wr + d_rhi_Wr], axis=1).T,
        (d_ant_bl + d_rhm_bl + d_rhi_bl)[None])
    out_news = _tc_matmul(
        A_news, jnp.concatenate([d_rant_Wl, d_rant_Wr], axis=1).T,
        d_rant_bl[None])

    return (out_tic, out_inst, out_mf, out_news)


# speedup vs baseline: 1.1632x; 1.1632x over previous
import jax
import jax.numpy as jnp
from jax import lax
from jax.experimental import pallas as pl
from jax.experimental.pallas import tpu as pltpu
from jax.experimental.pallas import tpu_sc as plsc

NC = 2
NS = 16
NW = NC * NS
EB = 128


def _ru(x, m):
    return (x + m - 1) // m * m


def _sc_segment_sums(tables, srcs, dsts, n_dst_pads, splits, zeros_f,
                     nbuf=2):
    nspec = len(tables)
    k0s = [s[0] for s in splits]
    k1s = [s[1] for s in splits]
    kmax = max(s.shape[1] for s in srcs)
    W = srcs[0].shape[2]

    out_types = [jax.ShapeDtypeStruct((NC, p, 128), jnp.float32)
                 for p in n_dst_pads]
    scratch = [pltpu.VMEM_SHARED((p, 128), jnp.float32) for p in n_dst_pads]
    scratch += [
        pltpu.VMEM((kmax, W), jnp.int32),
        pltpu.VMEM((kmax, W), jnp.int32),
        pltpu.VMEM((nbuf, W, 128), jnp.float32),
    ]
    scratch += [pltpu.SemaphoreType.DMA] * (2 * nbuf)
    mesh = plsc.VectorSubcoreMesh(core_axis_name="c", subcore_axis_name="s")

    def body(*refs):
        it = iter(refs)
        t_refs = [next(it) for _ in range(nspec)]
        s_refs = [next(it) for _ in range(nspec)]
        d_refs = [next(it) for _ in range(nspec)]
        zf_ref = next(it)
        o_refs = [next(it) for _ in range(nspec)]
        a_refs = [next(it) for _ in range(nspec)]
        sidx = next(it)
        didx = next(it)
        rows = next(it)
        gsem = [next(it) for _ in range(nbuf)]
        ssem = [next(it) for _ in range(nbuf)]

        cid = lax.axis_index("c")
        sid = lax.axis_index("s")
        wid = sid * NC + cid

        for a, p in zip(a_refs, n_dst_pads):
            st = p // NS
            pltpu.sync_copy(zf_ref.at[pl.ds(0, st), :],
                            a.at[pl.ds(sid * st, st), :])
        plsc.subcore_barrier()

        for si in range(nspec):
            kmx = max(k0s[si], k1s[si])
            pltpu.sync_copy(s_refs[si].at[wid], sidx.at[pl.ds(0, kmx), :])
            pltpu.sync_copy(d_refs[si].at[wid], didx.at[pl.ds(0, kmx), :])

            def run(k, si=si):
                gd = [None] * nbuf
                sd = [None] * nbuf
                for j in range(min(nbuf, k)):
                    gd[j] = pltpu.async_copy(t_refs[si].at[sidx.at[j]],
                                             rows.at[j], gsem[j])
                for j in range(k):
                    b = j % nbuf
                    gd[b].wait()
                    sd[b] = pltpu.async_copy(rows.at[b],
                                             a_refs[si].at[didx.at[j]],
                                             ssem[b], add=True)
                    jn = j + nbuf
                    if jn < k:
                        sd[b].wait()
                        gd[b] = pltpu.async_copy(
                            t_refs[si].at[sidx.at[jn]], rows.at[b], gsem[b])
                for j in range(max(0, k - nbuf), k):
                    sd[j % nbuf].wait()

            if k0s[si] == k1s[si]:
                run(k0s[si])
            else:
                @pl.when(cid == 0)
                def _(si=si):
                    run(k0s[si])

                @pl.when(cid != 0)
                def _(si=si):
                    run(k1s[si])
        plsc.subcore_barrier()

        for a, o, p in zip(a_refs, o_refs, n_dst_pads):
            st = p // NS
            pltpu.sync_copy(a.at[pl.ds(sid * st, st), :],
                            o.at[cid, pl.ds(sid * st, st), :])

    fn = pl.kernel(body, out_type=tuple(out_types), mesh=mesh,
                   scratch_types=scratch)
    return fn(*tables, *srcs, *dsts, zeros_f)


def _sc_segment_counts(dsts, n_dst_pads, zeros_f, ones_b):
    nspec = len(dsts)
    ks = [d.shape[1] for d in dsts]
    kmax = max(ks)
    W = dsts[0].shape[2]

    out_types = [jax.ShapeDtypeStruct((NC, p, 128), jnp.float32)
                 for p in n_dst_pads]
    scratch = [pltpu.VMEM_SHARED((p, 128), jnp.float32) for p in n_dst_pads]
    scratch += [
        pltpu.VMEM((nspec, kmax, W), jnp.int32),
        pltpu.VMEM((W, 128), jnp.float32),
        pltpu.SemaphoreType.DMA,
    ]
    mesh = plsc.VectorSubcoreMesh(core_axis_name="c", subcore_axis_name="s")

    def body(*refs):
        it = iter(refs)
        d_refs = [next(it) for _ in range(nspec)]
        zf_ref = next(it)
        on_ref = next(it)
        c_refs = [next(it) for _ in range(nspec)]
        ca_refs = [next(it) for _ in range(nspec)]
        didx = next(it)
        ones = next(it)
        sem = next(it)

        cid = lax.axis_index("c")
        sid = lax.axis_index("s")
        wid = sid * NC + cid

        for ca, p in zip(ca_refs, n_dst_pads):
            st = p // NS
            pltpu.sync_copy(zf_ref.at[pl.ds(0, st), :],
                            ca.at[pl.ds(sid * st, st), :])
        pltpu.sync_copy(on_ref, ones)
        plsc.subcore_barrier()

        for si in range(nspec):
            k = ks[si]
            pltpu.sync_copy(d_refs[si].at[wid],
                            didx.at[si, pl.ds(0, k), :])
        descs = []
        for si in range(nspec):
            for j in range(ks[si]):
                descs.append(pltpu.async_copy(
                    ones, ca_refs[si].at[didx.at[si, j]], sem, add=True))
        for d in descs:
            d.wait()
        plsc.subcore_barrier()

        for ca, co, p in zip(ca_refs, c_refs, n_dst_pads):
            st = p // NS
            pltpu.sync_copy(ca.at[pl.ds(sid * st, st), :],
                            co.at[cid, pl.ds(sid * st, st), :])

    fn = pl.kernel(body, out_type=tuple(out_types), mesh=mesh,
                   scratch_types=scratch)
    return fn(*dsts, zeros_f, ones_b)


def _pad_edges(ei, n_dst, f0=0.5):
    ne = ei.shape[1]
    ne_pad = _ru(ne, NW * EB)
    rtot = ne_pad // EB
    per16 = rtot // NS
    k0 = min(per16 - 1, max(1, int(round(per16 * f0))))
    k1 = per16 - k0
    kmax = max(k0, k1)

    def shape(x, fill):
        xp = jnp.concatenate(
            [x, jnp.full((ne_pad - ne,), fill, jnp.int32)]).reshape(rtot, EB)
        a0 = xp[:NS * k0].reshape(NS, k0, EB)
        a1 = xp[NS * k0:].reshape(NS, k1, EB)
        pad0 = jnp.full((NS, kmax - k0, EB), fill, jnp.int32)
        pad1 = jnp.full((NS, kmax - k1, EB), fill, jnp.int32)
        a0 = jnp.concatenate([a0, pad0], axis=1)
        a1 = jnp.concatenate([a1, pad1], axis=1)
        return jnp.stack([a0, a1], axis=1).reshape(NW, kmax, EB)

    return shape(ei[0], 0), shape(ei[1], n_dst), (k0, k1)


def _tc_matmul(A, W, b, bo=512):
    M, K = A.shape
    O = W.shape[1]
    bo = min(bo, O)

    def mm(a_ref, w_ref, b_ref, o_ref):
        o_ref[...] = jnp.dot(a_ref[...], w_ref[...],
                             preferred_element_type=jnp.float32) + b_ref[...]

    return pl.pallas_call(
        mm,
        grid=(pl.cdiv(O, bo),),
        in_specs=[pl.BlockSpec((M, K), lambda j: (0, 0)),
                  pl.BlockSpec((K, bo), lambda j: (0, j)),
                  pl.BlockSpec((1, bo), lambda j: (0, j))],
        out_specs=pl.BlockSpec((M, bo), lambda j: (0, j)),
        out_shape=jax.ShapeDtypeStruct((M, O), jnp.float32),
    )(A, W, b)


def _tc_combine(S_list, C_list, extra, n_out, pack):
    nsp = len(S_list)

    def body(*refs):
        it = iter(refs)
        Ss = [next(it) for _ in range(nsp)]
        Cs = [next(it) for _ in range(nsp)]
        e_ref = next(it)
        o_ref = next(it)
        acc = None
        for i, (S, C) in enumerate(zip(Ss, Cs)):
            sv = S[0] + S[1]
            cv = C[0, :, 0:1] + C[1, :, 0:1]
            mean = (sv / jnp.maximum(cv, 1.0))[:n_out, :]
            if pack:
                o_ref[:, 128 * i:128 * (i + 1)] = mean
            else:
                acc = mean if acc is None else acc + mean
        if pack:
            o_ref[:, 128 * nsp:] = e_ref[...]
        else:
            o_ref[...] = acc + e_ref[...]

    ow = 128 * (nsp + 1) if pack else 128
    return pl.pallas_call(
        body,
        out_shape=jax.ShapeDtypeStruct((n_out, ow), jnp.float32),
    )(*S_list, *C_list, extra)


def kernel(x_ticker, x_institution, x_mutual_fund, x_news,
           ei_hi, ei_hm, ei_ant, ei_rhm, ei_rhi, ei_rant,
           e1_hi_Wl, e1_hi_bl, e1_hi_Wr, d_hi_Wl, d_hi_bl, d_hi_Wr,
           e1_hm_Wl, e1_hm_bl, e1_hm_Wr, d_hm_Wl, d_hm_bl, d_hm_Wr,
           e1_ant_Wl, e1_ant_bl, e1_ant_Wr, d_ant_Wl, d_ant_bl, d_ant_Wr,
           e1_rhm_Wl, e1_rhm_bl, e1_rhm_Wr, d_rhm_Wl, d_rhm_bl, d_rhm_Wr,
           e1_rhi_Wl, e1_rhi_bl, e1_rhi_Wr, d_rhi_Wl, d_rhi_bl, d_rhi_Wr,
           e1_rant_Wl, e1_rant_bl, e1_rant_Wr,
           d_rant_Wl, d_rant_bl, d_rant_Wr):
    n_tic, n_inst, n_mf, n_news = (x_ticker.shape[0], x_institution.shape[0],
                                   x_mutual_fund.shape[0], x_news.shape[0])
    p_tic = _ru(n_tic + 1, NS * 8)
    p_inst = _ru(n_inst + 1, NS * 8)
    p_mf = _ru(n_mf + 1, NS * 8)
    p_news = _ru(n_news + 1, NS * 8)
    max_st = max(p_tic, p_inst, p_mf, p_news) // NS

    zeros_f = jnp.zeros((_ru(max_st, 8), 128), jnp.float32)
    ones_b = jnp.ones((EB, 128), jnp.float32)

    z384 = jnp.zeros((384,), jnp.float32)
    z128 = jnp.zeros((128,), jnp.float32)
    Pt = _tc_matmul(
        x_ticker,
        jnp.concatenate([e1_hi_Wl.T, e1_hm_Wl.T, e1_rant_Wl.T,
                         (e1_ant_Wr + e1_rhm_Wr + e1_rhi_Wr).T], axis=1),
        jnp.concatenate([z384, e1_ant_bl + e1_rhm_bl + e1_rhi_bl])[None])
    P_hi, P_hm, P_rant, base_tic = (Pt[:, :128], Pt[:, 128:256],
                                    Pt[:, 256:384], Pt[:, 384:])
    Pi = _tc_matmul(x_institution,
                    jnp.concatenate([e1_rhi_Wl.T, e1_hi_Wr.T], axis=1),
                    jnp.concatenate([z128, e1_hi_bl])[None])
    P_rhi, base_inst = Pi[:, :128], Pi[:, 128:]
    Pm = _tc_matmul(x_mutual_fund,
                    jnp.concatenate([e1_rhm_Wl.T, e1_hm_Wr.T], axis=1),
                    jnp.concatenate([z128, e1_hm_bl])[None])
    P_rhm, base_mf = Pm[:, :128], Pm[:, 128:]
    Pn = _tc_matmul(x_news,
                    jnp.concatenate([e1_ant_Wl.T, e1_rant_Wr.T], axis=1),
                    jnp.concatenate([z128, e1_rant_bl])[None])
    P_ant, base_news = Pn[:, :128], Pn[:, 128:]

    F0 = 0.5
    s_hi, d_hi, sp_hi = _pad_edges(ei_hi, n_inst, F0)
    s_hm, d_hm, sp_hm = _pad_edges(ei_hm, n_mf, F0)
    s_ant, d_ant, sp_ant = _pad_edges(ei_ant, n_tic, F0)
    s_rhm, d_rhm, sp_rhm = _pad_edges(ei_rhm, n_tic, F0)
    s_rhi, d_rhi, sp_rhi = _pad_edges(ei_rhi, n_tic, F0)
    s_rant, d_rant, sp_rant = _pad_edges(ei_rant, n_news, F0)

    (C_hi, C_hm, C_ant) = _sc_segment_counts(
        [d_hi, d_hm, d_ant], [p_inst, p_mf, p_tic], zeros_f, ones_b)
    (C_rhm, C_rhi, C_rant) = _sc_segment_counts(
        [d_rhm, d_rhi, d_rant], [p_tic, p_tic, p_news], zeros_f, ones_b)
    (S_hi, S_hm, S_ant) = _sc_segment_sums(
        [P_hi, P_hm, P_ant], [s_hi, s_hm, s_ant], [d_hi, d_hm, d_ant],
        [p_inst, p_mf, p_tic], [sp_hi, sp_hm, sp_ant], zeros_f, nbuf=3)
    (S_rhm, S_rhi, S_rant) = _sc_segment_sums(
        [P_rhm, P_rhi, P_rant], [s_rhm, s_rhi, s_rant],
        [d_rhm, d_rhi, d_rant], [p_tic, p_tic, p_news],
        [sp_rhm, sp_rhi, sp_rant], zeros_f, nbuf=3)

    z_inst = _tc_combine([S_hi], [C_hi], base_inst, n_inst, False)
    z_mf = _tc_combine([S_hm], [C_hm], base_mf, n_mf, False)
    z_tic = _tc_combine([S_ant, S_rhm, S_rhi], [C_ant, C_rhm, C_rhi],
                        base_tic, n_tic, False)
    z_news = _tc_combine([S_rant], [C_rant], base_news, n_news, False)

    (Sd_hi, Sd_hm, Sd_ant) = _sc_segment_sums(
        [z_tic, z_tic, z_news], [s_hi, s_hm, s_ant], [d_hi, d_hm, d_ant],
        [p_inst, p_mf, p_tic], [sp_hi, sp_hm, sp_ant], zeros_f, nbuf=3)
    (Sd_rhm, Sd_rhi, Sd_rant) = _sc_segment_sums(
        [z_mf, z_inst, z_tic], [s_rhm, s_rhi, s_rant],
        [d_rhm, d_rhi, d_rant], [p_tic, p_tic, p_news],
        [sp_rhm, sp_rhi, sp_rant], zeros_f, nbuf=3)

    A_inst = _tc_combine([Sd_hi], [C_hi], z_inst, n_inst, True)
    A_mf = _tc_combine([Sd_hm], [C_hm], z_mf, n_mf, True)
    A_tic = _tc_combine([Sd_ant, Sd_rhm, Sd_rhi], [C_ant, C_rhm, C_rhi],
                        z_tic, n_tic, True)
    A_news = _tc_combine([Sd_rant], [C_rant], z_news, n_news, True)

    out_inst = _tc_matmul(
        A_inst, jnp.concatenate([d_hi_Wl, d_hi_Wr], axis=1).T,
        d_hi_bl[None])
    out_mf = _tc_matmul(
        A_mf, jnp.concatenate([d_hm_Wl, d_hm_Wr], axis=1).T,
        d_hm_bl[None])
    out_tic = _tc_matmul(
        A_tic,
        jnp.concatenate([d_ant_Wl, d_rhm_Wl, d_rhi_Wl,
                         d_ant_Wr + d_rhm_Wr + d_rhi_Wr], axis=1).T,
        (d_ant_bl + d_rhm_bl + d_rhi_bl)[None])
    out_news = _tc_matmul(
        A_news, jnp.concatenate([d_rant_Wl, d_rant_Wr], axis=1).T,
        d_rant_bl[None])

    return (out_tic, out_inst, out_mf, out_news)


# speedup vs baseline: 2.2166x; 1.9057x over previous
import jax
import jax.numpy as jnp
from jax import lax
from jax.experimental import pallas as pl
from jax.experimental.pallas import tpu as pltpu
from jax.experimental.pallas import tpu_sc as plsc

NC = 2
NS = 16
NW = NC * NS
EB = 128


def _ru(x, m):
    return (x + m - 1) // m * m


def _sc_segment_sums(tables, srcs, dsts, n_dst_pads, splits, zeros_f,
                     nbuf=2, stage=False):
    nspec = len(tables)
    k0s = [s[0] for s in splits]
    k1s = [s[1] for s in splits]
    kmax = max(s.shape[1] for s in srcs)
    W = srcs[0].shape[2]

    out_types = [jax.ShapeDtypeStruct((NC, p, 128), jnp.float32)
                 for p in n_dst_pads]
    tps = [t.shape[0] for t in tables]
    if stage:
        assert all(tp % (NS * 8) == 0 for tp in tps)
    scratch = [pltpu.VMEM_SHARED((p, 128), jnp.float32) for p in n_dst_pads]
    if stage:
        scratch += [pltpu.VMEM_SHARED((tp, 128), jnp.float32) for tp in tps]
    scratch += [
        pltpu.VMEM((kmax, W), jnp.int32),
        pltpu.VMEM((kmax, W), jnp.int32),
        pltpu.VMEM((nbuf, W, 128), jnp.float32),
    ]
    scratch += [pltpu.SemaphoreType.DMA] * (2 * nbuf)
    mesh = plsc.VectorSubcoreMesh(core_axis_name="c", subcore_axis_name="s")

    def body(*refs):
        it = iter(refs)
        t_refs = [next(it) for _ in range(nspec)]
        s_refs = [next(it) for _ in range(nspec)]
        d_refs = [next(it) for _ in range(nspec)]
        zf_ref = next(it)
        o_refs = [next(it) for _ in range(nspec)]
        a_refs = [next(it) for _ in range(nspec)]
        st_refs = [next(it) for _ in range(nspec)] if stage else t_refs
        sidx = next(it)
        didx = next(it)
        rows = next(it)
        gsem = [next(it) for _ in range(nbuf)]
        ssem = [next(it) for _ in range(nbuf)]

        cid = lax.axis_index("c")
        sid = lax.axis_index("s")
        wid = sid * NC + cid

        for a, p in zip(a_refs, n_dst_pads):
            st = p // NS
            pltpu.sync_copy(zf_ref.at[pl.ds(0, st), :],
                            a.at[pl.ds(sid * st, st), :])
        if stage:
            for t, s_t, tp in zip(t_refs, st_refs, tps):
                stt = tp // NS
                pltpu.sync_copy(t.at[pl.ds(sid * stt, stt), :],
                                s_t.at[pl.ds(sid * stt, stt), :])
        plsc.subcore_barrier()

        for si in range(nspec):
            kmx = max(k0s[si], k1s[si])
            pltpu.sync_copy(s_refs[si].at[wid], sidx.at[pl.ds(0, kmx), :])
            pltpu.sync_copy(d_refs[si].at[wid], didx.at[pl.ds(0, kmx), :])

            def run(k, si=si):
                gd = [None] * nbuf
                sd = [None] * nbuf
                for j in range(min(nbuf, k)):
                    gd[j] = pltpu.async_copy(st_refs[si].at[sidx.at[j]],
                                             rows.at[j], gsem[j])
                for j in range(k):
                    b = j % nbuf
                    gd[b].wait()
                    sd[b] = pltpu.async_copy(rows.at[b],
                                             a_refs[si].at[didx.at[j]],
                                             ssem[b], add=True)
                    jn = j + nbuf
                    if jn < k:
                        sd[b].wait()
                        gd[b] = pltpu.async_copy(
                            st_refs[si].at[sidx.at[jn]], rows.at[b], gsem[b])
                for j in range(max(0, k - nbuf), k):
                    sd[j % nbuf].wait()

            if k0s[si] == k1s[si]:
                run(k0s[si])
            else:
                @pl.when(cid == 0)
                def _(si=si):
                    run(k0s[si])

                @pl.when(cid != 0)
                def _(si=si):
                    run(k1s[si])
        plsc.subcore_barrier()

        for a, o, p in zip(a_refs, o_refs, n_dst_pads):
            st = p // NS
            pltpu.sync_copy(a.at[pl.ds(sid * st, st), :],
                            o.at[cid, pl.ds(sid * st, st), :])

    fn = pl.kernel(body, out_type=tuple(out_types), mesh=mesh,
                   scratch_types=scratch)
    return fn(*tables, *srcs, *dsts, zeros_f)


def _sc_segment_counts(dsts, n_dst_pads, zeros_f, ones_b):
    nspec = len(dsts)
    ks = [d.shape[1] for d in dsts]
    kmax = max(ks)
    W = dsts[0].shape[2]

    out_types = [jax.ShapeDtypeStruct((NC, p, 128), jnp.float32)
                 for p in n_dst_pads]
    scratch = [pltpu.VMEM_SHARED((p, 128), jnp.float32) for p in n_dst_pads]
    scratch += [
        pltpu.VMEM((nspec, kmax, W), jnp.int32),
        pltpu.VMEM((W, 128), jnp.float32),
        pltpu.SemaphoreType.DMA,
    ]
    mesh = plsc.VectorSubcoreMesh(core_axis_name="c", subcore_axis_name="s")

    def body(*refs):
        it = iter(refs)
        d_refs = [next(it) for _ in range(nspec)]
        zf_ref = next(it)
        on_ref = next(it)
        c_refs = [next(it) for _ in range(nspec)]
        ca_refs = [next(it) for _ in range(nspec)]
        didx = next(it)
        ones = next(it)
        sem = next(it)

        cid = lax.axis_index("c")
        sid = lax.axis_index("s")
        wid = sid * NC + cid

        for ca, p in zip(ca_refs, n_dst_pads):
            st = p // NS
            pltpu.sync_copy(zf_ref.at[pl.ds(0, st), :],
                            ca.at[pl.ds(sid * st, st), :])
        pltpu.sync_copy(on_ref, ones)
        plsc.subcore_barrier()

        for si in range(nspec):
            k = ks[si]
            pltpu.sync_copy(d_refs[si].at[wid],
                            didx.at[si, pl.ds(0, k), :])
        descs = []
        for si in range(nspec):
            for j in range(ks[si]):
                descs.append(pltpu.async_copy(
                    ones, ca_refs[si].at[didx.at[si, j]], sem, add=True))
        for d in descs:
            d.wait()
        plsc.subcore_barrier()

        for ca, co, p in zip(ca_refs, c_refs, n_dst_pads):
            st = p // NS
            pltpu.sync_copy(ca.at[pl.ds(sid * st, st), :],
                            co.at[cid, pl.ds(sid * st, st), :])

    fn = pl.kernel(body, out_type=tuple(out_types), mesh=mesh,
                   scratch_types=scratch)
    return fn(*dsts, zeros_f, ones_b)


def _pad_edges(ei, n_dst, f0=0.5):
    ne = ei.shape[1]
    ne_pad = _ru(ne, NW * EB)
    rtot = ne_pad // EB
    per16 = rtot // NS
    k0 = min(per16 - 1, max(1, int(round(per16 * f0))))
    k1 = per16 - k0
    kmax = max(k0, k1)

    def shape(x, fill):
        xp = jnp.concatenate(
            [x, jnp.full((ne_pad - ne,), fill, jnp.int32)]).reshape(rtot, EB)
        a0 = xp[:NS * k0].reshape(NS, k0, EB)
        a1 = xp[NS * k0:].reshape(NS, k1, EB)
        pad0 = jnp.full((NS, kmax - k0, EB), fill, jnp.int32)
        pad1 = jnp.full((NS, kmax - k1, EB), fill, jnp.int32)
        a0 = jnp.concatenate([a0, pad0], axis=1)
        a1 = jnp.concatenate([a1, pad1], axis=1)
        return jnp.stack([a0, a1], axis=1).reshape(NW, kmax, EB)

    return shape(ei[0], 0), shape(ei[1], n_dst), (k0, k1)


def _tc_matmul(A, W, b, bo=512):
    M, K = A.shape
    O = W.shape[1]
    bo = min(bo, O)

    def mm(a_ref, w_ref, b_ref, o_ref):
        o_ref[...] = jnp.dot(a_ref[...], w_ref[...],
                             preferred_element_type=jnp.float32) + b_ref[...]

    return pl.pallas_call(
        mm,
        grid=(pl.cdiv(O, bo),),
        in_specs=[pl.BlockSpec((M, K), lambda j: (0, 0)),
                  pl.BlockSpec((K, bo), lambda j: (0, j)),
                  pl.BlockSpec((1, bo), lambda j: (0, j))],
        out_specs=pl.BlockSpec((M, bo), lambda j: (0, j)),
        out_shape=jax.ShapeDtypeStruct((M, O), jnp.float32),
    )(A, W, b)


def _tc_combine(S_list, C_list, extra, n_out, pack):
    nsp = len(S_list)

    def body(*refs):
        it = iter(refs)
        Ss = [next(it) for _ in range(nsp)]
        Cs = [next(it) for _ in range(nsp)]
        e_ref = next(it)
        o_ref = next(it)
        acc = None
        for i, (S, C) in enumerate(zip(Ss, Cs)):
            sv = S[0] + S[1]
            cv = C[0, :, 0:1] + C[1, :, 0:1]
            mean = (sv / jnp.maximum(cv, 1.0))[:n_out, :]
            if pack:
                o_ref[:, 128 * i:128 * (i + 1)] = mean
            else:
                acc = mean if acc is None else acc + mean
        if pack:
            o_ref[:, 128 * nsp:] = e_ref[...]
        else:
            o_ref[...] = acc + e_ref[...]

    ow = 128 * (nsp + 1) if pack else 128
    return pl.pallas_call(
        body,
        out_shape=jax.ShapeDtypeStruct((n_out, ow), jnp.float32),
    )(*S_list, *C_list, extra)


def kernel(x_ticker, x_institution, x_mutual_fund, x_news,
           ei_hi, ei_hm, ei_ant, ei_rhm, ei_rhi, ei_rant,
           e1_hi_Wl, e1_hi_bl, e1_hi_Wr, d_hi_Wl, d_hi_bl, d_hi_Wr,
           e1_hm_Wl, e1_hm_bl, e1_hm_Wr, d_hm_Wl, d_hm_bl, d_hm_Wr,
           e1_ant_Wl, e1_ant_bl, e1_ant_Wr, d_ant_Wl, d_ant_bl, d_ant_Wr,
           e1_rhm_Wl, e1_rhm_bl, e1_rhm_Wr, d_rhm_Wl, d_rhm_bl, d_rhm_Wr,
           e1_rhi_Wl, e1_rhi_bl, e1_rhi_Wr, d_rhi_Wl, d_rhi_bl, d_rhi_Wr,
           e1_rant_Wl, e1_rant_bl, e1_rant_Wr,
           d_rant_Wl, d_rant_bl, d_rant_Wr):
    n_tic, n_inst, n_mf, n_news = (x_ticker.shape[0], x_institution.shape[0],
                                   x_mutual_fund.shape[0], x_news.shape[0])
    p_tic = _ru(n_tic + 1, NS * 8)
    p_inst = _ru(n_inst + 1, NS * 8)
    p_mf = _ru(n_mf + 1, NS * 8)
    p_news = _ru(n_news + 1, NS * 8)
    max_st = max(p_tic, p_inst, p_mf, p_news) // NS

    zeros_f = jnp.zeros((_ru(max_st, 8), 128), jnp.float32)
    ones_b = jnp.ones((EB, 128), jnp.float32)

    z384 = jnp.zeros((384,), jnp.float32)
    z128 = jnp.zeros((128,), jnp.float32)
    Pt = _tc_matmul(
        x_ticker,
        jnp.concatenate([e1_hi_Wl.T, e1_hm_Wl.T, e1_rant_Wl.T,
                         (e1_ant_Wr + e1_rhm_Wr + e1_rhi_Wr).T], axis=1),
        jnp.concatenate([z384, e1_ant_bl + e1_rhm_bl + e1_rhi_bl])[None])
    P_hi, P_hm, P_rant, base_tic = (Pt[:, :128], Pt[:, 128:256],
                                    Pt[:, 256:384], Pt[:, 384:])
    Pi = _tc_matmul(x_institution,
                    jnp.concatenate([e1_rhi_Wl.T, e1_hi_Wr.T], axis=1),
                    jnp.concatenate([z128, e1_hi_bl])[None])
    P_rhi, base_inst = Pi[:, :128], Pi[:, 128:]
    Pm = _tc_matmul(x_mutual_fund,
                    jnp.concatenate([e1_rhm_Wl.T, e1_hm_Wr.T], axis=1),
                    jnp.concatenate([z128, e1_hm_bl])[None])
    P_rhm, base_mf = Pm[:, :128], Pm[:, 128:]
    Pn = _tc_matmul(x_news,
                    jnp.concatenate([e1_ant_Wl.T, e1_rant_Wr.T], axis=1),
                    jnp.concatenate([z128, e1_rant_bl])[None])
    P_ant, base_news = Pn[:, :128], Pn[:, 128:]

    F0 = 0.5
    s_hi, d_hi, sp_hi = _pad_edges(ei_hi, n_inst, F0)
    s_hm, d_hm, sp_hm = _pad_edges(ei_hm, n_mf, F0)
    s_ant, d_ant, sp_ant = _pad_edges(ei_ant, n_tic, F0)
    s_rhm, d_rhm, sp_rhm = _pad_edges(ei_rhm, n_tic, F0)
    s_rhi, d_rhi, sp_rhi = _pad_edges(ei_rhi, n_tic, F0)
    s_rant, d_rant, sp_rant = _pad_edges(ei_rant, n_news, F0)

    (C_hi, C_hm, C_ant) = _sc_segment_counts(
        [d_hi, d_hm, d_ant], [p_inst, p_mf, p_tic], zeros_f, ones_b)
    (C_rhm, C_rhi, C_rant) = _sc_segment_counts(
        [d_rhm, d_rhi, d_rant], [p_tic, p_tic, p_news], zeros_f, ones_b)
    def pad128(t):
        return jnp.pad(t, ((0, _ru(t.shape[0], 128) - t.shape[0]), (0, 0)))

    def agg(table, s3, d3, p, sp):
        (out,) = _sc_segment_sums([pad128(table)], [s3], [d3], [p], [sp],
                                  zeros_f, nbuf=2, stage=True)
        return out

    S_hi = agg(P_hi, s_hi, d_hi, p_inst, sp_hi)
    S_hm = agg(P_hm, s_hm, d_hm, p_mf, sp_hm)
    S_ant = agg(P_ant, s_ant, d_ant, p_tic, sp_ant)
    S_rhm = agg(P_rhm, s_rhm, d_rhm, p_tic, sp_rhm)
    S_rhi = agg(P_rhi, s_rhi, d_rhi, p_tic, sp_rhi)
    S_rant = agg(P_rant, s_rant, d_rant, p_news, sp_rant)

    z_inst = _tc_combine([S_hi], [C_hi], base_inst, n_inst, False)
    z_mf = _tc_combine([S_hm], [C_hm], base_mf, n_mf, False)
    z_tic = _tc_combine([S_ant, S_rhm, S_rhi], [C_ant, C_rhm, C_rhi],
                        base_tic, n_tic, False)
    z_news = _tc_combine([S_rant], [C_rant], base_news, n_news, False)

    Sd_hi = agg(z_tic, s_hi, d_hi, p_inst, sp_hi)
    Sd_hm = agg(z_tic, s_hm, d_hm, p_mf, sp_hm)
    Sd_ant = agg(z_news, s_ant, d_ant, p_tic, sp_ant)
    Sd_rhm = agg(z_mf, s_rhm, d_rhm, p_tic, sp_rhm)
    Sd_rhi = agg(z_inst, s_rhi, d_rhi, p_tic, sp_rhi)
    Sd_rant = agg(z_tic, s_rant, d_rant, p_news, sp_rant)

    A_inst = _tc_combine([Sd_hi], [C_hi], z_inst, n_inst, True)
    A_mf = _tc_combine([Sd_hm], [C_hm], z_mf, n_mf, True)
    A_tic = _tc_combine([Sd_ant, Sd_rhm, Sd_rhi], [C_ant, C_rhm, C_rhi],
                        z_tic, n_tic, True)
    A_news = _tc_combine([Sd_rant], [C_rant], z_news, n_news, True)

    out_inst = _tc_matmul(
        A_inst, jnp.concatenate([d_hi_Wl, d_hi_Wr], axis=1).T,
        d_hi_bl[None])
    out_mf = _tc_matmul(
        A_mf, jnp.concatenate([d_hm_Wl, d_hm_Wr], axis=1).T,
        d_hm_bl[None])
    out_tic = _tc_matmul(
        A_tic,
        jnp.concatenate([d_ant_Wl, d_rhm_Wl, d_rhi_Wl,
                         d_ant_Wr + d_rhm_Wr + d_rhi_Wr], axis=1).T,
        (d_ant_bl + d_rhm_bl + d_rhi_bl)[None])
    out_news = _tc_matmul(
        A_news, jnp.concatenate([d_rant_Wl, d_rant_Wr], axis=1).T,
        d_rant_bl[None])

    return (out_tic, out_inst, out_mf, out_news)


# speedup vs baseline: 2.2190x; 1.0011x over previous
import jax
import jax.numpy as jnp
from jax import lax
from jax.experimental import pallas as pl
from jax.experimental.pallas import tpu as pltpu
from jax.experimental.pallas import tpu_sc as plsc

NC = 2
NS = 16
NW = NC * NS
EB = 128


def _ru(x, m):
    return (x + m - 1) // m * m


def _sc_segment_sums(tables, srcs, dsts, n_dst_pads, splits, zeros_f,
                     nbuf=2, stage=False):
    nspec = len(tables)
    k0s = [s[0] for s in splits]
    k1s = [s[1] for s in splits]
    kmax = max(s.shape[1] for s in srcs)
    W = srcs[0].shape[2]

    out_types = [jax.ShapeDtypeStruct((NC, p, 128), jnp.float32)
                 for p in n_dst_pads]
    tps = [t.shape[0] for t in tables]
    if stage:
        assert all(tp % (NS * 8) == 0 for tp in tps)
    scratch = [pltpu.VMEM_SHARED((p, 128), jnp.float32) for p in n_dst_pads]
    if stage:
        scratch += [pltpu.VMEM_SHARED((tp, 128), jnp.float32) for tp in tps]
    scratch += [
        pltpu.VMEM((kmax, W), jnp.int32),
        pltpu.VMEM((kmax, W), jnp.int32),
        pltpu.VMEM((nbuf, W, 128), jnp.float32),
    ]
    scratch += [pltpu.SemaphoreType.DMA] * (2 * nbuf)
    mesh = plsc.VectorSubcoreMesh(core_axis_name="c", subcore_axis_name="s")

    def body(*refs):
        it = iter(refs)
        t_refs = [next(it) for _ in range(nspec)]
        s_refs = [next(it) for _ in range(nspec)]
        d_refs = [next(it) for _ in range(nspec)]
        zf_ref = next(it)
        o_refs = [next(it) for _ in range(nspec)]
        a_refs = [next(it) for _ in range(nspec)]
        st_refs = [next(it) for _ in range(nspec)] if stage else t_refs
        sidx = next(it)
        didx = next(it)
        rows = next(it)
        gsem = [next(it) for _ in range(nbuf)]
        ssem = [next(it) for _ in range(nbuf)]

        cid = lax.axis_index("c")
        sid = lax.axis_index("s")
        wid = sid * NC + cid

        for a, p in zip(a_refs, n_dst_pads):
            st = p // NS
            pltpu.sync_copy(zf_ref.at[pl.ds(0, st), :],
                            a.at[pl.ds(sid * st, st), :])
        if stage:
            for t, s_t, tp in zip(t_refs, st_refs, tps):
                stt = tp // NS
                pltpu.sync_copy(t.at[pl.ds(sid * stt, stt), :],
                                s_t.at[pl.ds(sid * stt, stt), :])
        plsc.subcore_barrier()

        for si in range(nspec):
            kmx = max(k0s[si], k1s[si])
            pltpu.sync_copy(s_refs[si].at[wid], sidx.at[pl.ds(0, kmx), :])
            pltpu.sync_copy(d_refs[si].at[wid], didx.at[pl.ds(0, kmx), :])

            def run(k, si=si):
                gd = [None] * nbuf
                sd = [None] * nbuf
                for j in range(min(nbuf, k)):
                    gd[j] = pltpu.async_copy(st_refs[si].at[sidx.at[j]],
                                             rows.at[j], gsem[j])
                for j in range(k):
                    b = j % nbuf
                    gd[b].wait()
                    sd[b] = pltpu.async_copy(rows.at[b],
                                             a_refs[si].at[didx.at[j]],
                                             ssem[b], add=True)
                    jn = j + nbuf
                    if jn < k:
                        sd[b].wait()
                        gd[b] = pltpu.async_copy(
                            st_refs[si].at[sidx.at[jn]], rows.at[b], gsem[b])
                for j in range(max(0, k - nbuf), k):
                    sd[j % nbuf].wait()

            if k0s[si] == k1s[si]:
                run(k0s[si])
            else:
                @pl.when(cid == 0)
                def _(si=si):
                    run(k0s[si])

                @pl.when(cid != 0)
                def _(si=si):
                    run(k1s[si])
        plsc.subcore_barrier()

        for a, o, p in zip(a_refs, o_refs, n_dst_pads):
            st = p // NS
            pltpu.sync_copy(a.at[pl.ds(sid * st, st), :],
                            o.at[cid, pl.ds(sid * st, st), :])

    fn = pl.kernel(body, out_type=tuple(out_types), mesh=mesh,
                   scratch_types=scratch)
    return fn(*tables, *srcs, *dsts, zeros_f)


def _sc_segment_counts(dsts, n_dst_pads, zeros_f, ones_b):
    nspec = len(dsts)
    ks = [d.shape[1] for d in dsts]
    kmax = max(ks)
    W = dsts[0].shape[2]

    out_types = [jax.ShapeDtypeStruct((NC, p, 128), jnp.float32)
                 for p in n_dst_pads]
    scratch = [pltpu.VMEM_SHARED((p, 128), jnp.float32) for p in n_dst_pads]
    scratch += [
        pltpu.VMEM((nspec, kmax, W), jnp.int32),
        pltpu.VMEM((W, 128), jnp.float32),
        pltpu.SemaphoreType.DMA,
    ]
    mesh = plsc.VectorSubcoreMesh(core_axis_name="c", subcore_axis_name="s")

    def body(*refs):
        it = iter(refs)
        d_refs = [next(it) for _ in range(nspec)]
        zf_ref = next(it)
        on_ref = next(it)
        c_refs = [next(it) for _ in range(nspec)]
        ca_refs = [next(it) for _ in range(nspec)]
        didx = next(it)
        ones = next(it)
        sem = next(it)

        cid = lax.axis_index("c")
        sid = lax.axis_index("s")
        wid = sid * NC + cid

        for ca, p in zip(ca_refs, n_dst_pads):
            st = p // NS
            pltpu.sync_copy(zf_ref.at[pl.ds(0, st), :],
                            ca.at[pl.ds(sid * st, st), :])
        pltpu.sync_copy(on_ref, ones)
        plsc.subcore_barrier()

        for si in range(nspec):
            k = ks[si]
            pltpu.sync_copy(d_refs[si].at[wid],
                            didx.at[si, pl.ds(0, k), :])
        descs = []
        for si in range(nspec):
            for j in range(ks[si]):
                descs.append(pltpu.async_copy(
                    ones, ca_refs[si].at[didx.at[si, j]], sem, add=True))
        for d in descs:
            d.wait()
        plsc.subcore_barrier()

        for ca, co, p in zip(ca_refs, c_refs, n_dst_pads):
            st = p // NS
            pltpu.sync_copy(ca.at[pl.ds(sid * st, st), :],
                            co.at[cid, pl.ds(sid * st, st), :])

    fn = pl.kernel(body, out_type=tuple(out_types), mesh=mesh,
                   scratch_types=scratch)
    return fn(*dsts, zeros_f, ones_b)


def _pad_edges(ei, n_dst, f0=0.5):
    ne = ei.shape[1]
    ne_pad = _ru(ne, NW * EB)
    rtot = ne_pad // EB
    per16 = rtot // NS
    k0 = min(per16 - 1, max(1, int(round(per16 * f0))))
    k1 = per16 - k0
    kmax = max(k0, k1)

    def shape(x, fill):
        xp = jnp.concatenate(
            [x, jnp.full((ne_pad - ne,), fill, jnp.int32)]).reshape(rtot, EB)
        a0 = xp[:NS * k0].reshape(NS, k0, EB)
        a1 = xp[NS * k0:].reshape(NS, k1, EB)
        pad0 = jnp.full((NS, kmax - k0, EB), fill, jnp.int32)
        pad1 = jnp.full((NS, kmax - k1, EB), fill, jnp.int32)
        a0 = jnp.concatenate([a0, pad0], axis=1)
        a1 = jnp.concatenate([a1, pad1], axis=1)
        return jnp.stack([a0, a1], axis=1).reshape(NW, kmax, EB)

    return shape(ei[0], 0), shape(ei[1], n_dst), (k0, k1)


def _tc_matmul(A, W, b, bo=512):
    M, K = A.shape
    O = W.shape[1]
    bo = min(bo, O)

    def mm(a_ref, w_ref, b_ref, o_ref):
        o_ref[...] = jnp.dot(a_ref[...], w_ref[...],
                             preferred_element_type=jnp.float32) + b_ref[...]

    return pl.pallas_call(
        mm,
        grid=(pl.cdiv(O, bo),),
        in_specs=[pl.BlockSpec((M, K), lambda j: (0, 0)),
                  pl.BlockSpec((K, bo), lambda j: (0, j)),
                  pl.BlockSpec((1, bo), lambda j: (0, j))],
        out_specs=pl.BlockSpec((M, bo), lambda j: (0, j)),
        out_shape=jax.ShapeDtypeStruct((M, O), jnp.float32),
    )(A, W, b)


def _tc_combine(S_list, C_list, extra, n_out, pack):
    nsp = len(S_list)

    def body(*refs):
        it = iter(refs)
        Ss = [next(it) for _ in range(nsp)]
        Cs = [next(it) for _ in range(nsp)]
        e_ref = next(it)
        o_ref = next(it)
        acc = None
        for i, (S, C) in enumerate(zip(Ss, Cs)):
            sv = S[0] + S[1]
            cv = C[0, :, 0:1] + C[1, :, 0:1]
            mean = (sv / jnp.maximum(cv, 1.0))[:n_out, :]
            if pack:
                o_ref[:, 128 * i:128 * (i + 1)] = mean
            else:
                acc = mean if acc is None else acc + mean
        if pack:
            o_ref[:, 128 * nsp:] = e_ref[...]
        else:
            o_ref[...] = acc + e_ref[...]

    ow = 128 * (nsp + 1) if pack else 128
    return pl.pallas_call(
        body,
        out_shape=jax.ShapeDtypeStruct((n_out, ow), jnp.float32),
    )(*S_list, *C_list, extra)


def kernel(x_ticker, x_institution, x_mutual_fund, x_news,
           ei_hi, ei_hm, ei_ant, ei_rhm, ei_rhi, ei_rant,
           e1_hi_Wl, e1_hi_bl, e1_hi_Wr, d_hi_Wl, d_hi_bl, d_hi_Wr,
           e1_hm_Wl, e1_hm_bl, e1_hm_Wr, d_hm_Wl, d_hm_bl, d_hm_Wr,
           e1_ant_Wl, e1_ant_bl, e1_ant_Wr, d_ant_Wl, d_ant_bl, d_ant_Wr,
           e1_rhm_Wl, e1_rhm_bl, e1_rhm_Wr, d_rhm_Wl, d_rhm_bl, d_rhm_Wr,
           e1_rhi_Wl, e1_rhi_bl, e1_rhi_Wr, d_rhi_Wl, d_rhi_bl, d_rhi_Wr,
           e1_rant_Wl, e1_rant_bl, e1_rant_Wr,
           d_rant_Wl, d_rant_bl, d_rant_Wr):
    n_tic, n_inst, n_mf, n_news = (x_ticker.shape[0], x_institution.shape[0],
                                   x_mutual_fund.shape[0], x_news.shape[0])
    p_tic = _ru(n_tic + 1, NS * 8)
    p_inst = _ru(n_inst + 1, NS * 8)
    p_mf = _ru(n_mf + 1, NS * 8)
    p_news = _ru(n_news + 1, NS * 8)
    max_st = max(p_tic, p_inst, p_mf, p_news) // NS

    zeros_f = jnp.zeros((_ru(max_st, 8), 128), jnp.float32)
    ones_b = jnp.ones((EB, 128), jnp.float32)

    z384 = jnp.zeros((384,), jnp.float32)
    z128 = jnp.zeros((128,), jnp.float32)
    Pt = _tc_matmul(
        x_ticker,
        jnp.concatenate([e1_hi_Wl.T, e1_hm_Wl.T, e1_rant_Wl.T,
                         (e1_ant_Wr + e1_rhm_Wr + e1_rhi_Wr).T], axis=1),
        jnp.concatenate([z384, e1_ant_bl + e1_rhm_bl + e1_rhi_bl])[None])
    P_hi, P_hm, P_rant, base_tic = (Pt[:, :128], Pt[:, 128:256],
                                    Pt[:, 256:384], Pt[:, 384:])
    Pi = _tc_matmul(x_institution,
                    jnp.concatenate([e1_rhi_Wl.T, e1_hi_Wr.T], axis=1),
                    jnp.concatenate([z128, e1_hi_bl])[None])
    P_rhi, base_inst = Pi[:, :128], Pi[:, 128:]
    Pm = _tc_matmul(x_mutual_fund,
                    jnp.concatenate([e1_rhm_Wl.T, e1_hm_Wr.T], axis=1),
                    jnp.concatenate([z128, e1_hm_bl])[None])
    P_rhm, base_mf = Pm[:, :128], Pm[:, 128:]
    Pn = _tc_matmul(x_news,
                    jnp.concatenate([e1_ant_Wl.T, e1_rant_Wr.T], axis=1),
                    jnp.concatenate([z128, e1_rant_bl])[None])
    P_ant, base_news = Pn[:, :128], Pn[:, 128:]

    F0 = 0.5
    s_hi, d_hi, sp_hi = _pad_edges(ei_hi, n_inst, F0)
    s_hm, d_hm, sp_hm = _pad_edges(ei_hm, n_mf, F0)
    s_ant, d_ant, sp_ant = _pad_edges(ei_ant, n_tic, F0)
    s_rhm, d_rhm, sp_rhm = _pad_edges(ei_rhm, n_tic, F0)
    s_rhi, d_rhi, sp_rhi = _pad_edges(ei_rhi, n_tic, F0)
    s_rant, d_rant, sp_rant = _pad_edges(ei_rant, n_news, F0)

    (C_hi, C_hm, C_ant) = _sc_segment_counts(
        [d_hi, d_hm, d_ant], [p_inst, p_mf, p_tic], zeros_f, ones_b)
    (C_rhm, C_rhi, C_rant) = _sc_segment_counts(
        [d_rhm, d_rhi, d_rant], [p_tic, p_tic, p_news], zeros_f, ones_b)
    def pad128(t):
        return jnp.pad(t, ((0, _ru(t.shape[0], 128) - t.shape[0]), (0, 0)))

    def agg(table, s3, d3, p, sp):
        (out,) = _sc_segment_sums([pad128(table)], [s3], [d3], [p], [sp],
                                  zeros_f, nbuf=3, stage=True)
        return out

    S_hi = agg(P_hi, s_hi, d_hi, p_inst, sp_hi)
    S_hm = agg(P_hm, s_hm, d_hm, p_mf, sp_hm)
    S_ant = agg(P_ant, s_ant, d_ant, p_tic, sp_ant)
    S_rhm = agg(P_rhm, s_rhm, d_rhm, p_tic, sp_rhm)
    S_rhi = agg(P_rhi, s_rhi, d_rhi, p_tic, sp_rhi)
    S_rant = agg(P_rant, s_rant, d_rant, p_news, sp_rant)

    z_inst = _tc_combine([S_hi], [C_hi], base_inst, n_inst, False)
    z_mf = _tc_combine([S_hm], [C_hm], base_mf, n_mf, False)
    z_tic = _tc_combine([S_ant, S_rhm, S_rhi], [C_ant, C_rhm, C_rhi],
                        base_tic, n_tic, False)
    z_news = _tc_combine([S_rant], [C_rant], base_news, n_news, False)

    Sd_hi = agg(z_tic, s_hi, d_hi, p_inst, sp_hi)
    Sd_hm = agg(z_tic, s_hm, d_hm, p_mf, sp_hm)
    Sd_ant = agg(z_news, s_ant, d_ant, p_tic, sp_ant)
    Sd_rhm = agg(z_mf, s_rhm, d_rhm, p_tic, sp_rhm)
    Sd_rhi = agg(z_inst, s_rhi, d_rhi, p_tic, sp_rhi)
    Sd_rant = agg(z_tic, s_rant, d_rant, p_news, sp_rant)

    A_inst = _tc_combine([Sd_hi], [C_hi], z_inst, n_inst, True)
    A_mf = _tc_combine([Sd_hm], [C_hm], z_mf, n_mf, True)
    A_tic = _tc_combine([Sd_ant, Sd_rhm, Sd_rhi], [C_ant, C_rhm, C_rhi],
                        z_tic, n_tic, True)
    A_news = _tc_combine([Sd_rant], [C_rant], z_news, n_news, True)

    out_inst = _tc_matmul(
        A_inst, jnp.concatenate([d_hi_Wl, d_hi_Wr], axis=1).T,
        d_hi_bl[None])
    out_mf = _tc_matmul(
        A_mf, jnp.concatenate([d_hm_Wl, d_hm_Wr], axis=1).T,
        d_hm_bl[None])
    out_tic = _tc_matmul(
        A_tic,
        jnp.concatenate([d_ant_Wl, d_rhm_Wl, d_rhi_Wl,
                         d_ant_Wr + d_rhm_Wr + d_rhi_Wr], axis=1).T,
        (d_ant_bl + d_rhm_bl + d_rhi_bl)[None])
    out_news = _tc_matmul(
        A_news, jnp.concatenate([d_rant_Wl, d_rant_Wr], axis=1).T,
        d_rant_bl[None])

    return (out_tic, out_inst, out_mf, out_news)
